# initial kernel scaffold (unmeasured)
import functools

import jax
import jax.numpy as jnp
from jax import lax
from jax.experimental import pallas as pl
from jax.experimental.pallas import tpu as pltpu

N_DEV = 4
SQ = 2048
SKV = 2048
DM = 1024
HL = 8
DH = 128
DL = HL * DH
QB = 256
KW = 768
WIN = 128
SCALE = 0.08838834764831843


def kernel(x, Wq, K_ext, V_ext, Wo):
    my = lax.axis_index("i")
    xb = x[0].astype(jnp.bfloat16)
    wq_my = lax.dynamic_slice_in_dim(Wq, my * DL, DL, axis=1)
    wq_my = wq_my.astype(jnp.bfloat16)
    wo_my = lax.dynamic_slice_in_dim(Wo, my * DL, DL, axis=0)
    wo_my = wo_my.astype(jnp.bfloat16)
    kb = K_ext[0].transpose(1, 0, 2).astype(jnp.bfloat16)
    vb = V_ext[0].transpose(1, 0, 2).astype(jnp.bfloat16)

    def body(x_ref, wq_ref, k_ref, v_ref, wo_ref, out_ref,
             q_ref, ctx_ref, comm_ref, send_sems, recv_sems):
        my_pos = lax.axis_index("i")
        left = lax.rem(my_pos + N_DEV - 1, N_DEV)
        right = lax.rem(my_pos + 1, N_DEV)

        barrier_sem = pltpu.get_barrier_semaphore()
        for nbr in (left, right):
            pl.semaphore_signal(barrier_sem, inc=1, device_id=(nbr,),
                                device_id_type=pl.DeviceIdType.MESH)
        pl.semaphore_wait(barrier_sem, 2)

        q_ref[...] = lax.dot_general(
            x_ref[...], wq_ref[...], (((1,), (0,)), ((), ())),
            preferred_element_type=jnp.float32).astype(jnp.bfloat16)

        for h in range(HL):
            for qb in range(SQ // QB):
                s = min(max(QB * qb - 256, 0), SKV - KW)
                qblk = q_ref[pl.ds(qb * QB, QB), pl.ds(h * DH, DH)]
                kwin = k_ref[h, pl.ds(s, KW), :]
                vwin = v_ref[h, pl.ds(s, KW), :]
                scores = lax.dot_general(
                    qblk, kwin, (((1,), (1,)), ((), ())),
                    preferred_element_type=jnp.float32) * SCALE
                qi = lax.broadcasted_iota(jnp.int32, (QB, KW), 0) + qb * QB
                kj = lax.broadcasted_iota(jnp.int32, (QB, KW), 1) + s
                mask = jnp.abs(qi - kj) <= WIN
                scores = jnp.where(mask, scores, -1e9)
                m = jnp.max(scores, axis=1, keepdims=True)
                w = jnp.exp(scores - m)
                w = w / jnp.sum(w, axis=1, keepdims=True)
                ctx_blk = lax.dot_general(
                    w.astype(jnp.bfloat16), vwin, (((1,), (0,)), ((), ())),
                    preferred_element_type=jnp.float32)
                ctx_ref[pl.ds(qb * QB, QB), pl.ds(h * DH, DH)] = (
                    ctx_blk.astype(jnp.bfloat16))

        partial = lax.dot_general(
            ctx_ref[...], wo_ref[...], (((1,), (0,)), ((), ())),
            preferred_element_type=jnp.float32)
        out_ref[0, :, :] = partial
        comm_ref[0, :, :] = partial.astype(jnp.bfloat16)

        for hop in range(N_DEV - 1):
            rdma = pltpu.make_async_remote_copy(
                src_ref=comm_ref.at[hop],
                dst_ref=comm_ref.at[hop + 1],
                send_sem=send_sems.at[hop],
                recv_sem=recv_sems.at[hop],
                device_id=(right,),
                device_id_type=pl.DeviceIdType.MESH,
            )
            rdma.start()
            rdma.wait()
            out_ref[0, :, :] = (out_ref[0, :, :]
                                + comm_ref[hop + 1].astype(jnp.float32))

        @functools.partial(pl.run_scoped, sem2=pltpu.SemaphoreType.REGULAR)
        def _(sem2):
            for nbr in (left, right):
                pl.semaphore_signal(sem2, inc=1, device_id=(nbr,),
                                    device_id_type=pl.DeviceIdType.MESH)
            pl.semaphore_wait(sem2, 2)

    return pl.pallas_call(
        body,
        out_shape=jax.ShapeDtypeStruct((1, SQ, DM), jnp.float32),
        in_specs=[pl.BlockSpec(memory_space=pltpu.VMEM)] * 5,
        out_specs=pl.BlockSpec(memory_space=pltpu.VMEM),
        scratch_shapes=[
            pltpu.VMEM((SQ, DL), jnp.bfloat16),
            pltpu.VMEM((SQ, DL), jnp.bfloat16),
            pltpu.VMEM((N_DEV, SQ, DM), jnp.bfloat16),
            pltpu.SemaphoreType.DMA((N_DEV - 1,)),
            pltpu.SemaphoreType.DMA((N_DEV - 1,)),
        ],
        compiler_params=pltpu.CompilerParams(collective_id=0),
    )(xb, wq_my, kb, vb, wo_my)


# baseline (device time: 230575 ns/iter reference)
import functools

import jax
import jax.numpy as jnp
from jax import lax
from jax.experimental import pallas as pl
from jax.experimental.pallas import tpu as pltpu

N_DEV = 4
SQ = 2048
SKV = 2048
DM = 1024
HL = 8
DH = 128
DL = HL * DH
QB = 256
KW = 768
WIN = 128
SCALE = 0.08838834764831843


def kernel(x, Wq, K_ext, V_ext, Wo):
    my = lax.axis_index("i")
    xb = x[0].astype(jnp.bfloat16)
    wq_my = lax.dynamic_slice_in_dim(Wq, my * DL, DL, axis=1)
    wq_my = wq_my.astype(jnp.bfloat16)
    wo_my = lax.dynamic_slice_in_dim(Wo, my * DL, DL, axis=0)
    wo_my = wo_my.astype(jnp.bfloat16)
    kb = K_ext[0].transpose(1, 0, 2).astype(jnp.bfloat16)
    vb = V_ext[0].transpose(1, 0, 2).astype(jnp.bfloat16)

    def body(x_ref, wq_ref, k_ref, v_ref, wo_ref, out_ref,
             q_ref, ctx_ref, comm_ref, send_sems, recv_sems):
        my_pos = lax.axis_index("i")
        left = lax.rem(my_pos + N_DEV - 1, N_DEV)
        right = lax.rem(my_pos + 1, N_DEV)

        barrier_sem = pltpu.get_barrier_semaphore()
        for nbr in (left, right):
            pl.semaphore_signal(barrier_sem, inc=1, device_id=(nbr,),
                                device_id_type=pl.DeviceIdType.MESH)
        pl.semaphore_wait(barrier_sem, 2)

        for qb in range(SQ // QB):
            q_ref[pl.ds(qb * QB, QB), :] = lax.dot_general(
                x_ref[pl.ds(qb * QB, QB), :], wq_ref[...],
                (((1,), (0,)), ((), ())),
                preferred_element_type=jnp.float32).astype(jnp.bfloat16)

        for h in range(HL):
            for qb in range(SQ // QB):
                s = min(max(QB * qb - 256, 0), SKV - KW)
                qblk = q_ref[pl.ds(qb * QB, QB), pl.ds(h * DH, DH)]
                kwin = k_ref[h, pl.ds(s, KW), :]
                vwin = v_ref[h, pl.ds(s, KW), :]
                scores = lax.dot_general(
                    qblk, kwin, (((1,), (1,)), ((), ())),
                    preferred_element_type=jnp.float32) * SCALE
                qi = lax.broadcasted_iota(jnp.int32, (QB, KW), 0) + qb * QB
                kj = lax.broadcasted_iota(jnp.int32, (QB, KW), 1) + s
                mask = jnp.abs(qi - kj) <= WIN
                scores = jnp.where(mask, scores, -1e9)
                m = jnp.max(scores, axis=1, keepdims=True)
                w = jnp.exp(scores - m)
                w = w / jnp.sum(w, axis=1, keepdims=True)
                ctx_blk = lax.dot_general(
                    w.astype(jnp.bfloat16), vwin, (((1,), (0,)), ((), ())),
                    preferred_element_type=jnp.float32)
                ctx_ref[pl.ds(qb * QB, QB), pl.ds(h * DH, DH)] = (
                    ctx_blk.astype(jnp.bfloat16))

        for qb in range(SQ // QB):
            partial = lax.dot_general(
                ctx_ref[pl.ds(qb * QB, QB), :], wo_ref[...],
                (((1,), (0,)), ((), ())),
                preferred_element_type=jnp.float32)
            out_ref[0, pl.ds(qb * QB, QB), :] = partial
            comm_ref[0, pl.ds(qb * QB, QB), :] = partial.astype(jnp.bfloat16)

        for hop in range(N_DEV - 1):
            rdma = pltpu.make_async_remote_copy(
                src_ref=comm_ref.at[hop],
                dst_ref=comm_ref.at[hop + 1],
                send_sem=send_sems.at[hop],
                recv_sem=recv_sems.at[hop],
                device_id=(right,),
                device_id_type=pl.DeviceIdType.MESH,
            )
            rdma.start()
            rdma.wait()
            for qb in range(SQ // QB):
                blk = pl.ds(qb * QB, QB)
                out_ref[0, blk, :] = (
                    out_ref[0, blk, :]
                    + comm_ref[hop + 1, blk, :].astype(jnp.float32))

        @functools.partial(pl.run_scoped, sem2=pltpu.SemaphoreType.REGULAR)
        def _(sem2):
            for nbr in (left, right):
                pl.semaphore_signal(sem2, inc=1, device_id=(nbr,),
                                    device_id_type=pl.DeviceIdType.MESH)
            pl.semaphore_wait(sem2, 2)

    return pl.pallas_call(
        body,
        out_shape=jax.ShapeDtypeStruct((1, SQ, DM), jnp.float32),
        in_specs=[pl.BlockSpec(memory_space=pltpu.VMEM)] * 5,
        out_specs=pl.BlockSpec(memory_space=pltpu.VMEM),
        scratch_shapes=[
            pltpu.VMEM((SQ, DL), jnp.bfloat16),
            pltpu.VMEM((SQ, DL), jnp.bfloat16),
            pltpu.VMEM((N_DEV, SQ, DM), jnp.bfloat16),
            pltpu.SemaphoreType.DMA((N_DEV - 1,)),
            pltpu.SemaphoreType.DMA((N_DEV - 1,)),
        ],
        compiler_params=pltpu.CompilerParams(
            collective_id=0,
            vmem_limit_bytes=100 * 1024 * 1024,
        ),
    )(xb, wq_my, kb, vb, wo_my)


# device time: 134639 ns/iter; 1.7125x vs baseline; 1.7125x over previous
import functools

import jax
import jax.numpy as jnp
from jax import lax
from jax.experimental import pallas as pl
from jax.experimental.pallas import tpu as pltpu

N_DEV = 4
SQ = 2048
SKV = 2048
DM = 1024
HL = 8
DH = 128
DL = HL * DH
QB = 256
KW = 768
WIN = 128
SCALE = 0.08838834764831843


def kernel(x, Wq, K_ext, V_ext, Wo):
    my = lax.axis_index("i")
    xb = x[0].astype(jnp.bfloat16)
    wq_my = lax.dynamic_slice_in_dim(Wq, my * DL, DL, axis=1)
    wq_my = wq_my.astype(jnp.bfloat16)
    wo_my = lax.dynamic_slice_in_dim(Wo, my * DL, DL, axis=0)
    wo_my = wo_my.astype(jnp.bfloat16)
    kb = K_ext[0].transpose(1, 0, 2).astype(jnp.bfloat16)
    vb = V_ext[0].transpose(1, 0, 2).astype(jnp.bfloat16)

    def body(x_ref, wq_ref, k_ref, v_ref, wo_ref, out_ref,
             q_ref, ctx_ref,
             rs_sendR, rs_recvR, ag_sendR, ag_recvR,
             rs_sendL, rs_recvL, ag_sendL, ag_recvL,
             sendR_sems, recvR_sems, sendL_sems, recvL_sems):
        my_pos = lax.axis_index("i")
        left = lax.rem(my_pos + N_DEV - 1, N_DEV)
        right = lax.rem(my_pos + 1, N_DEV)

        barrier_sem = pltpu.get_barrier_semaphore()
        for nbr in (left, right):
            pl.semaphore_signal(barrier_sem, inc=1, device_id=(nbr,),
                                device_id_type=pl.DeviceIdType.MESH)
        pl.semaphore_wait(barrier_sem, 2)

        for qb in range(SQ // QB):
            q_ref[pl.ds(qb * QB, QB), :] = lax.dot_general(
                x_ref[pl.ds(qb * QB, QB), :], wq_ref[...],
                (((1,), (0,)), ((), ())),
                preferred_element_type=jnp.float32).astype(jnp.bfloat16)

        for h in range(HL):
            for qb in range(SQ // QB):
                s = min(max(QB * qb - 256, 0), SKV - KW)
                qblk = q_ref[pl.ds(qb * QB, QB), pl.ds(h * DH, DH)]
                kwin = k_ref[h, pl.ds(s, KW), :]
                vwin = v_ref[h, pl.ds(s, KW), :]
                scores = lax.dot_general(
                    qblk, kwin, (((1,), (1,)), ((), ())),
                    preferred_element_type=jnp.float32) * SCALE
                qi = lax.broadcasted_iota(jnp.int32, (QB, KW), 0) + qb * QB
                kj = lax.broadcasted_iota(jnp.int32, (QB, KW), 1) + s
                mask = jnp.abs(qi - kj) <= WIN
                scores = jnp.where(mask, scores, -1e9)
                m = jnp.max(scores, axis=1, keepdims=True)
                w = jnp.exp(scores - m)
                w = w / jnp.sum(w, axis=1, keepdims=True)
                ctx_blk = lax.dot_general(
                    w.astype(jnp.bfloat16), vwin, (((1,), (0,)), ((), ())),
                    preferred_element_type=jnp.float32)
                ctx_ref[pl.ds(qb * QB, QB), pl.ds(h * DH, DH)] = (
                    ctx_blk.astype(jnp.bfloat16))

        for qb in range(SQ // QB):
            partial = lax.dot_general(
                ctx_ref[pl.ds(qb * QB, QB), :], wo_ref[...],
                (((1,), (0,)), ((), ())),
                preferred_element_type=jnp.float32)
            out_ref[0, pl.ds(qb * QB, QB), :] = partial

        QROWS = SQ // N_DEV

        def qmod(c):
            return lax.rem(my_pos + c + 2 * N_DEV, N_DEV)

        def rowA(q):
            return pl.ds(q * QROWS, QB)

        def rowB(q):
            return pl.ds(q * QROWS + QB, QB)

        def copy(src, dst, s_sems, r_sems, idx, dev):
            return pltpu.make_async_remote_copy(
                src_ref=src, dst_ref=dst,
                send_sem=s_sems.at[idx], recv_sem=r_sems.at[idx],
                device_id=(dev,), device_id_type=pl.DeviceIdType.MESH)

        rs_sendR[0] = out_ref[0, rowA(qmod(0)), :].astype(jnp.bfloat16)
        rs_sendL[0] = out_ref[0, rowB(qmod(0)), :].astype(jnp.bfloat16)
        for t in range(N_DEV - 1):
            rR = copy(rs_sendR.at[t], rs_recvR.at[t],
                      sendR_sems, recvR_sems, t, right)
            rL = copy(rs_sendL.at[t], rs_recvL.at[t],
                      sendL_sems, recvL_sems, t, left)
            rR.start()
            rL.start()
            rR.wait()
            rL.wait()
            qR = qmod(-t - 1)
            qL = qmod(t + 1)
            accR = rs_recvR[t].astype(jnp.float32) + out_ref[0, rowA(qR), :]
            accL = rs_recvL[t].astype(jnp.float32) + out_ref[0, rowB(qL), :]
            if t < N_DEV - 2:
                rs_sendR[t + 1] = accR.astype(jnp.bfloat16)
                rs_sendL[t + 1] = accL.astype(jnp.bfloat16)
            else:
                out_ref[0, rowA(qR), :] = accR
                out_ref[0, rowB(qL), :] = accL
                ag_sendR[0] = accR.astype(jnp.bfloat16)
                ag_sendL[0] = accL.astype(jnp.bfloat16)

        for t in range(N_DEV - 1):
            srcR = ag_sendR.at[0] if t == 0 else ag_recvR.at[t - 1]
            srcL = ag_sendL.at[0] if t == 0 else ag_recvL.at[t - 1]
            aR = copy(srcR, ag_recvR.at[t],
                      sendR_sems, recvR_sems, (N_DEV - 1) + t, right)
            aL = copy(srcL, ag_recvL.at[t],
                      sendL_sems, recvL_sems, (N_DEV - 1) + t, left)
            aR.start()
            aL.start()
            aR.wait()
            aL.wait()
            out_ref[0, rowA(qmod(-t)), :] = ag_recvR[t].astype(jnp.float32)
            out_ref[0, rowB(qmod(t)), :] = ag_recvL[t].astype(jnp.float32)

        @functools.partial(pl.run_scoped, sem2=pltpu.SemaphoreType.REGULAR)
        def _(sem2):
            for nbr in (left, right):
                pl.semaphore_signal(sem2, inc=1, device_id=(nbr,),
                                    device_id_type=pl.DeviceIdType.MESH)
            pl.semaphore_wait(sem2, 2)

    return pl.pallas_call(
        body,
        out_shape=jax.ShapeDtypeStruct((1, SQ, DM), jnp.float32),
        in_specs=[pl.BlockSpec(memory_space=pltpu.VMEM)] * 5,
        out_specs=pl.BlockSpec(memory_space=pltpu.VMEM),
        scratch_shapes=[
            pltpu.VMEM((SQ, DL), jnp.bfloat16),
            pltpu.VMEM((SQ, DL), jnp.bfloat16),
            pltpu.VMEM((3, QB, DM), jnp.bfloat16),
            pltpu.VMEM((3, QB, DM), jnp.bfloat16),
            pltpu.VMEM((1, QB, DM), jnp.bfloat16),
            pltpu.VMEM((3, QB, DM), jnp.bfloat16),
            pltpu.VMEM((3, QB, DM), jnp.bfloat16),
            pltpu.VMEM((3, QB, DM), jnp.bfloat16),
            pltpu.VMEM((1, QB, DM), jnp.bfloat16),
            pltpu.VMEM((3, QB, DM), jnp.bfloat16),
            pltpu.SemaphoreType.DMA((6,)),
            pltpu.SemaphoreType.DMA((6,)),
            pltpu.SemaphoreType.DMA((6,)),
            pltpu.SemaphoreType.DMA((6,)),
        ],
        compiler_params=pltpu.CompilerParams(
            collective_id=0,
            vmem_limit_bytes=100 * 1024 * 1024,
        ),
    )(xb, wq_my, kb, vb, wo_my)


# device time: 100293 ns/iter; 2.2990x vs baseline; 1.3425x over previous
import functools

import jax
import jax.numpy as jnp
from jax import lax
from jax.experimental import pallas as pl
from jax.experimental.pallas import tpu as pltpu

N_DEV = 4
SQ = 2048
SKV = 2048
DM = 1024
HL = 8
DH = 128
DL = HL * DH
QB = 256
KW = 512
WIN = 128
QROWS = SQ // N_DEV
SCALE = 0.08838834764831843


def kernel(x, Wq, K_ext, V_ext, Wo):
    my = lax.axis_index("i")
    xb = x[0].astype(jnp.bfloat16)
    wq_my = lax.dynamic_slice_in_dim(Wq, my * DL, DL, axis=1)
    wq_my = wq_my.astype(jnp.bfloat16)
    wo_my = lax.dynamic_slice_in_dim(Wo, my * DL, DL, axis=0)
    wo_my = wo_my.astype(jnp.bfloat16)
    kb = K_ext[0].transpose(1, 0, 2).astype(jnp.bfloat16)
    vb = V_ext[0].transpose(1, 0, 2).astype(jnp.bfloat16)

    def body(x_ref, wq_ref, k_ref, v_ref, wo_ref, out_ref,
             q_ref, ctx_ref,
             rs_sendR, rs_recvR, ag_sendR, ag_recvR,
             rs_sendL, rs_recvL, ag_sendL, ag_recvL,
             sendR_sems, recvR_sems, sendL_sems, recvL_sems):
        my_pos = lax.axis_index("i")
        left = lax.rem(my_pos + N_DEV - 1, N_DEV)
        right = lax.rem(my_pos + 1, N_DEV)

        barrier_sem = pltpu.get_barrier_semaphore()
        for nbr in (left, right):
            pl.semaphore_signal(barrier_sem, inc=1, device_id=(nbr,),
                                device_id_type=pl.DeviceIdType.MESH)
        pl.semaphore_wait(barrier_sem, 2)

        def compute_quarter(qtr):
            for j in range(QROWS // QB):
                row = qtr * QROWS + j * QB
                rows = pl.ds(row, QB)
                q_ref[rows, :] = lax.dot_general(
                    x_ref[rows, :], wq_ref[...], (((1,), (0,)), ((), ())),
                    preferred_element_type=jnp.float32).astype(jnp.bfloat16)
                s = jnp.minimum(jnp.maximum(row - 128, 0), SKV - KW)
                s = (s // 128) * 128
                qi = lax.broadcasted_iota(jnp.int32, (QB, KW), 0) + row
                kj = lax.broadcasted_iota(jnp.int32, (QB, KW), 1) + s
                mask = jnp.abs(qi - kj) <= WIN
                for h in range(HL):
                    hcols = pl.ds(h * DH, DH)
                    qblk = q_ref[rows, hcols]
                    kwin = k_ref[h, pl.ds(s, KW), :]
                    vwin = v_ref[h, pl.ds(s, KW), :]
                    scores = lax.dot_general(
                        qblk, kwin, (((1,), (1,)), ((), ())),
                        preferred_element_type=jnp.float32) * SCALE
                    scores = jnp.where(mask, scores, -1e9)
                    w = jnp.exp(scores)
                    recip = 1.0 / jnp.sum(w, axis=1, keepdims=True)
                    ctx_blk = lax.dot_general(
                        w.astype(jnp.bfloat16), vwin, (((1,), (0,)), ((), ())),
                        preferred_element_type=jnp.float32) * recip
                    ctx_ref[rows, hcols] = ctx_blk.astype(jnp.bfloat16)
                out_ref[0, rows, :] = lax.dot_general(
                    ctx_ref[rows, :], wo_ref[...], (((1,), (0,)), ((), ())),
                    preferred_element_type=jnp.float32)

        def qmod(c):
            return lax.rem(my_pos + c + 2 * N_DEV, N_DEV)

        def rowA(q):
            return pl.ds(q * QROWS, QB)

        def rowB(q):
            return pl.ds(q * QROWS + QB, QB)

        def copy(src, dst, s_sems, r_sems, idx, dev):
            return pltpu.make_async_remote_copy(
                src_ref=src, dst_ref=dst,
                send_sem=s_sems.at[idx], recv_sem=r_sems.at[idx],
                device_id=(dev,), device_id_type=pl.DeviceIdType.MESH)

        def rs_hop(t):
            rR = copy(rs_sendR.at[t], rs_recvR.at[t],
                      sendR_sems, recvR_sems, t, right)
            rL = copy(rs_sendL.at[t], rs_recvL.at[t],
                      sendL_sems, recvL_sems, t, left)
            rR.start()
            rL.start()
            return rR, rL

        def rs_accum(t, rR, rL):
            rR.wait()
            rL.wait()
            qR = qmod(-t - 1)
            qL = qmod(t + 1)
            accR = rs_recvR[t].astype(jnp.float32) + out_ref[0, rowA(qR), :]
            accL = rs_recvL[t].astype(jnp.float32) + out_ref[0, rowB(qL), :]
            if t < N_DEV - 2:
                rs_sendR[t + 1] = accR.astype(jnp.bfloat16)
                rs_sendL[t + 1] = accL.astype(jnp.bfloat16)
            else:
                out_ref[0, rowA(qR), :] = accR
                out_ref[0, rowB(qL), :] = accL
                ag_sendR[0] = accR.astype(jnp.bfloat16)
                ag_sendL[0] = accL.astype(jnp.bfloat16)

        compute_quarter(qmod(0))
        rs_sendR[0] = out_ref[0, rowA(qmod(0)), :].astype(jnp.bfloat16)
        rs_sendL[0] = out_ref[0, rowB(qmod(0)), :].astype(jnp.bfloat16)
        h0 = rs_hop(0)
        compute_quarter(qmod(-1))
        compute_quarter(qmod(1))
        rs_accum(0, *h0)
        h1 = rs_hop(1)
        compute_quarter(qmod(2))
        rs_accum(1, *h1)
        h2 = rs_hop(2)
        rs_accum(2, *h2)

        for t in range(N_DEV - 1):
            srcR = ag_sendR.at[0] if t == 0 else ag_recvR.at[t - 1]
            srcL = ag_sendL.at[0] if t == 0 else ag_recvL.at[t - 1]
            aR = copy(srcR, ag_recvR.at[t],
                      sendR_sems, recvR_sems, (N_DEV - 1) + t, right)
            aL = copy(srcL, ag_recvL.at[t],
                      sendL_sems, recvL_sems, (N_DEV - 1) + t, left)
            aR.start()
            aL.start()
            aR.wait()
            aL.wait()
            out_ref[0, rowA(qmod(-t)), :] = ag_recvR[t].astype(jnp.float32)
            out_ref[0, rowB(qmod(t)), :] = ag_recvL[t].astype(jnp.float32)

        @functools.partial(pl.run_scoped, sem2=pltpu.SemaphoreType.REGULAR)
        def _(sem2):
            for nbr in (left, right):
                pl.semaphore_signal(sem2, inc=1, device_id=(nbr,),
                                    device_id_type=pl.DeviceIdType.MESH)
            pl.semaphore_wait(sem2, 2)

    return pl.pallas_call(
        body,
        out_shape=jax.ShapeDtypeStruct((1, SQ, DM), jnp.float32),
        in_specs=[pl.BlockSpec(memory_space=pltpu.VMEM)] * 5,
        out_specs=pl.BlockSpec(memory_space=pltpu.VMEM),
        scratch_shapes=[
            pltpu.VMEM((SQ, DL), jnp.bfloat16),
            pltpu.VMEM((SQ, DL), jnp.bfloat16),
            pltpu.VMEM((3, QB, DM), jnp.bfloat16),
            pltpu.VMEM((3, QB, DM), jnp.bfloat16),
            pltpu.VMEM((1, QB, DM), jnp.bfloat16),
            pltpu.VMEM((3, QB, DM), jnp.bfloat16),
            pltpu.VMEM((3, QB, DM), jnp.bfloat16),
            pltpu.VMEM((3, QB, DM), jnp.bfloat16),
            pltpu.VMEM((1, QB, DM), jnp.bfloat16),
            pltpu.VMEM((3, QB, DM), jnp.bfloat16),
            pltpu.SemaphoreType.DMA((6,)),
            pltpu.SemaphoreType.DMA((6,)),
            pltpu.SemaphoreType.DMA((6,)),
            pltpu.SemaphoreType.DMA((6,)),
        ],
        compiler_params=pltpu.CompilerParams(
            collective_id=0,
            vmem_limit_bytes=100 * 1024 * 1024,
        ),
    )(xb, wq_my, kb, vb, wo_my)
